# baseline (speedup 1.0000x reference)
import dataclasses
import functools

import jax
import jax.numpy as jnp
from jax import lax
from jax.experimental import pallas as pl
from jax.experimental.pallas import tpu as pltpu
from jax.experimental.pallas import tpu_sc as plsc

_GROUPS = (
    (0, 1, 2, 8, 25),
    (3, 24),
    (6, 7),
    (4, 9),
    (12, 13, 14, 15),
    (16, 17, 18, 19, 20, 21, 22, 23),
)

_N_COLS = 26
_N_OUT = 6
_LANES = 16
_ROWS_PER_BLOCK = 1600


def _block_body(in_vmem, out_vmem):
    lanes = lax.iota(jnp.int32, _LANES)
    row_in = lanes * _N_COLS
    row_out = lanes * _N_OUT

    @pl.loop(0, _ROWS_PER_BLOCK // _LANES)
    def _(i):
        b_in = row_in + i * (_LANES * _N_COLS)
        b_out = row_out + i * (_LANES * _N_OUT)
        cache = {}

        def col(c):
            if c not in cache:
                cache[c] = plsc.load_gather(in_vmem, [b_in + c])
            return cache[c]

        for g, idx in enumerate(_GROUPS):
            m = col(idx[0])
            for c in idx[1:]:
                m = jnp.maximum(m, col(c))
            plsc.store_scatter(out_vmem, [b_out + g], m)


def kernel(activation):
    n_rows = activation.shape[0]
    n_blocks = n_rows // _ROWS_PER_BLOCK
    flat = activation.reshape(-1)
    mesh = plsc.VectorSubcoreMesh(core_axis_name="c", subcore_axis_name="s")
    cp = pltpu.CompilerParams()
    if "needs_layout_passes" in pltpu.CompilerParams.__dataclass_fields__:
        cp = dataclasses.replace(cp, needs_layout_passes=False)

    @functools.partial(
        pl.kernel,
        out_type=jax.ShapeDtypeStruct((n_rows * _N_OUT,), jnp.float32),
        mesh=mesh,
        compiler_params=cp,
    )
    def run(in_hbm, out_hbm):
        pltpu.emit_pipeline(
            _block_body,
            grid=(n_blocks,),
            in_specs=[
                pl.BlockSpec((_ROWS_PER_BLOCK * _N_COLS,), lambda i: (i,))
            ],
            out_specs=[
                pl.BlockSpec((_ROWS_PER_BLOCK * _N_OUT,), lambda i: (i,))
            ],
            core_axis_name=("c", "s"),
            dimension_semantics=(pltpu.PARALLEL,),
        )(in_hbm, out_hbm)

    return run(flat).reshape(n_rows, _N_OUT)
